# Initial kernel scaffold; baseline (speedup 1.0000x reference)
import jax
import jax.numpy as jnp
from jax.experimental import pallas as pl


def kernel(x, table, W1, b1, W2):
    raise NotImplementedError("write your pallas kernel here")


# speedup vs baseline: 1.5688x; 1.5688x over previous
import functools

import jax
import jax.numpy as jnp
from jax import lax
from jax.experimental import pallas as pl
from jax.experimental.pallas import tpu as pltpu
from jax.experimental.pallas import tpu_sc as plsc

_NUM_CORES = 2
_NUM_SUBCORES = 16
_NW = _NUM_CORES * _NUM_SUBCORES
_CHUNK = 128


def _sc_gather(table, idx3d, per_w, n_chunks):
    _, d = table.shape
    b_total = _NW * per_w
    mesh = plsc.VectorSubcoreMesh(core_axis_name="c", subcore_axis_name="s")

    n_groups = 4
    cpg = n_chunks // n_groups
    rows_pg = cpg * _CHUNK

    @functools.partial(
        pl.kernel,
        out_type=jax.ShapeDtypeStruct((b_total, d), jnp.float32),
        mesh=mesh,
        scratch_types=[
            pltpu.VMEM((n_chunks, _CHUNK), jnp.int32),
            pltpu.VMEM((2, rows_pg, d), jnp.float32),
            pltpu.SemaphoreType.DMA,
            pltpu.SemaphoreType.DMA,
        ],
        compiler_params=pltpu.CompilerParams(use_tc_tiling_on_sc=False),
    )
    def gather_kernel(table_hbm, idx_hbm, out_hbm, idx_v, rows_v, sem0, sem1):
        wid = lax.axis_index("s") * _NUM_CORES + lax.axis_index("c")
        base = wid * per_w
        sems = (sem0, sem1)
        pltpu.sync_copy(idx_hbm.at[wid], idx_v)

        def fire(g):
            buf = rows_v.at[g % 2]
            return [
                pltpu.async_copy(
                    table_hbm.at[idx_v.at[g * cpg + c]],
                    buf.at[pl.ds(c * _CHUNK, _CHUNK)],
                    sems[g % 2],
                )
                for c in range(cpg)
            ]

        def drain_store(g, copies):
            for cp in copies:
                cp.wait()
            pltpu.sync_copy(
                rows_v.at[g % 2], out_hbm.at[pl.ds(base + g * rows_pg, rows_pg)]
            )

        for g in range(n_groups):
            drain_store(g, fire(g))

    return gather_kernel(table, idx3d)


def _mlp(win, W1, b1, W2):
    b, k = win.shape
    h = W1.shape[0]
    o = W2.shape[0]
    blk = 2048

    def body(win_ref, w1_ref, b1_ref, w2_ref, out_ref):
        z = lax.dot_general(
            win_ref[...], w1_ref[...],
            (((1,), (1,)), ((), ())),
            preferred_element_type=jnp.float32,
        )
        act = jax.nn.sigmoid(z + b1_ref[...])
        out_ref[...] = lax.dot_general(
            act, w2_ref[...],
            (((1,), (1,)), ((), ())),
            preferred_element_type=jnp.float32,
        )

    return pl.pallas_call(
        body,
        grid=(b // blk,),
        in_specs=[
            pl.BlockSpec((blk, k), lambda i: (i, 0)),
            pl.BlockSpec((h, k), lambda i: (0, 0)),
            pl.BlockSpec((1, h), lambda i: (0, 0)),
            pl.BlockSpec((o, h), lambda i: (0, 0)),
        ],
        out_specs=pl.BlockSpec((blk, o), lambda i: (i, 0)),
        out_shape=jax.ShapeDtypeStruct((b, o), jnp.float32),
    )(win, W1, b1.reshape(1, h), W2)


def kernel(x, table, W1, b1, W2):
    batch, win = x.shape
    d = table.shape[1]
    n_idx = batch * win
    per_w = n_idx // _NW
    n_chunks = per_w // _CHUNK
    idx3d = x.astype(jnp.int32).reshape(_NW, n_chunks, _CHUNK)
    gathered = _sc_gather(table, idx3d, per_w, n_chunks)
    win_emb = gathered.reshape(batch, win * d)
    return _mlp(win_emb, W1, b1, W2)


# speedup vs baseline: 1.5696x; 1.0005x over previous
import functools

import jax
import jax.numpy as jnp
from jax import lax
from jax.experimental import pallas as pl
from jax.experimental.pallas import tpu as pltpu
from jax.experimental.pallas import tpu_sc as plsc

_NUM_CORES = 2
_NUM_SUBCORES = 16
_NW = _NUM_CORES * _NUM_SUBCORES
_CHUNK = 128


def _sc_gather(table, idx3d, per_w, n_chunks):
    _, d = table.shape
    b_total = _NW * per_w
    mesh = plsc.VectorSubcoreMesh(core_axis_name="c", subcore_axis_name="s")

    n_groups = 4
    cpg = n_chunks // n_groups
    rows_pg = cpg * _CHUNK

    @functools.partial(
        pl.kernel,
        out_type=jax.ShapeDtypeStruct((b_total, d), jnp.float32),
        mesh=mesh,
        scratch_types=[
            pltpu.VMEM((n_chunks, _CHUNK), jnp.int32),
            pltpu.VMEM((2, rows_pg, d), jnp.float32),
            pltpu.SemaphoreType.DMA,
            pltpu.SemaphoreType.DMA,
        ],
        compiler_params=pltpu.CompilerParams(use_tc_tiling_on_sc=False),
    )
    def gather_kernel(table_hbm, idx_hbm, out_hbm, idx_v, rows_v, sem0, sem1):
        wid = lax.axis_index("s") * _NUM_CORES + lax.axis_index("c")
        base = wid * per_w
        sems = (sem0, sem1)
        pltpu.sync_copy(idx_hbm.at[wid], idx_v)

        def fire(g):
            buf = rows_v.at[g % 2]
            return [
                pltpu.async_copy(
                    table_hbm.at[idx_v.at[g * cpg + c]],
                    buf.at[pl.ds(c * _CHUNK, _CHUNK)],
                    sems[g % 2],
                )
                for c in range(cpg)
            ]

        def drain_store(g, copies):
            for cp in copies:
                cp.wait()
            pltpu.sync_copy(
                rows_v.at[g % 2], out_hbm.at[pl.ds(base + g * rows_pg, rows_pg)]
            )

        prev = fire(0)
        for g in range(1, n_groups):
            cur = fire(g)
            drain_store(g - 1, prev)
            prev = cur
        drain_store(n_groups - 1, prev)

    return gather_kernel(table, idx3d)


def _mlp(win, W1, b1, W2):
    b, k = win.shape
    h = W1.shape[0]
    o = W2.shape[0]
    blk = 2048

    def body(win_ref, w1_ref, b1_ref, w2_ref, out_ref):
        z = lax.dot_general(
            win_ref[...], w1_ref[...],
            (((1,), (1,)), ((), ())),
            preferred_element_type=jnp.float32,
        )
        act = jax.nn.sigmoid(z + b1_ref[...])
        out_ref[...] = lax.dot_general(
            act, w2_ref[...],
            (((1,), (1,)), ((), ())),
            preferred_element_type=jnp.float32,
        )

    return pl.pallas_call(
        body,
        grid=(b // blk,),
        in_specs=[
            pl.BlockSpec((blk, k), lambda i: (i, 0)),
            pl.BlockSpec((h, k), lambda i: (0, 0)),
            pl.BlockSpec((1, h), lambda i: (0, 0)),
            pl.BlockSpec((o, h), lambda i: (0, 0)),
        ],
        out_specs=pl.BlockSpec((blk, o), lambda i: (i, 0)),
        out_shape=jax.ShapeDtypeStruct((b, o), jnp.float32),
    )(win, W1, b1.reshape(1, h), W2)


def kernel(x, table, W1, b1, W2):
    batch, win = x.shape
    d = table.shape[1]
    d_pad = 56
    n_idx = batch * win
    per_w = n_idx // _NW
    n_chunks = per_w // _CHUNK
    idx3d = x.astype(jnp.int32).reshape(_NW, n_chunks, _CHUNK)
    table_p = jnp.pad(table, ((0, 0), (0, d_pad - d)))
    gathered = _sc_gather(table_p, idx3d, per_w, n_chunks)
    win_emb = gathered[:, :d].reshape(batch, win * d)
    return _mlp(win_emb, W1, b1, W2)


# speedup vs baseline: 1.8192x; 1.1591x over previous
import functools

import jax
import jax.numpy as jnp
from jax import lax
from jax.experimental import pallas as pl
from jax.experimental.pallas import tpu as pltpu
from jax.experimental.pallas import tpu_sc as plsc

_NUM_CORES = 2
_NUM_SUBCORES = 16
_NW = _NUM_CORES * _NUM_SUBCORES
_CHUNK = 128
_NCHUNKS = 20
_BLK = 200
_D = 50
_SPG = 4


def _sc_gather_extract(table4, q3d, sb3d):
    n_rows_out = _NW * _NCHUNKS * _CHUNK * _D // 128
    mesh = plsc.VectorSubcoreMesh(core_axis_name="c", subcore_axis_name="s")
    stage_words = _SPG * _CHUNK * _D

    @functools.partial(
        pl.kernel,
        out_type=jax.ShapeDtypeStruct((n_rows_out, 128), jnp.float32),
        mesh=mesh,
        scratch_types=[
            pltpu.VMEM((_NCHUNKS, _CHUNK), jnp.int32),
            pltpu.VMEM((_NCHUNKS, _CHUNK), jnp.int32),
            pltpu.VMEM((_CHUNK, _BLK), jnp.float32),
            pltpu.VMEM((stage_words // 128, 128), jnp.float32),
            pltpu.SemaphoreType.DMA,
        ],
        compiler_params=pltpu.CompilerParams(
            use_tc_tiling_on_sc=False, needs_layout_passes=False
        ),
    )
    def gather_kernel(table_hbm, q_hbm, sb_hbm, out_hbm, q_v, sb_v, rows_v,
                      stage_v, sem):
        wid = lax.axis_index("s") * _NUM_CORES + lax.axis_index("c")
        pltpu.sync_copy(q_hbm.at[wid], q_v)
        pltpu.sync_copy(sb_hbm.at[wid], sb_v)
        lane = lax.iota(jnp.int32, 16)
        lane50 = lane * _D

        def chunk_body(ch, carry):
            pltpu.async_copy(table_hbm.at[q_v.at[ch]], rows_v, sem).wait()
            st_base = (ch % _SPG) * (_CHUNK * _D)
            for j in range(_CHUNK // 16):
                row_vec = lane + 16 * j
                col0 = sb_v[ch, pl.ds(16 * j, 16)]
                dst0 = st_base + (16 * j) * _D + lane50
                for k in range(_D):
                    v = plsc.load_gather(rows_v, [row_vec, col0 + k])
                    dst = dst0 + k
                    plsc.store_scatter(stage_v, [dst >> 7, dst & 127], v)
            @pl.when(ch % _SPG == _SPG - 1)
            def _():
                g = ch // _SPG
                off = wid * (_NCHUNKS * _CHUNK * _D // 128) + g * (stage_words // 128)
                pltpu.sync_copy(
                    stage_v, out_hbm.at[pl.ds(off, stage_words // 128)]
                )
            return carry

        lax.fori_loop(0, _NCHUNKS, chunk_body, None)

    return gather_kernel(table4, q3d, sb3d)


def _mlp(win, W1, b1, W2):
    b, k = win.shape
    h = W1.shape[0]
    o = W2.shape[0]
    blk = 2048

    def body(win_ref, w1_ref, b1_ref, w2_ref, out_ref):
        z = lax.dot_general(
            win_ref[...], w1_ref[...],
            (((1,), (1,)), ((), ())),
            preferred_element_type=jnp.float32,
        )
        act = jax.nn.sigmoid(z + b1_ref[...])
        out_ref[...] = lax.dot_general(
            act, w2_ref[...],
            (((1,), (1,)), ((), ())),
            preferred_element_type=jnp.float32,
        )

    return pl.pallas_call(
        body,
        grid=(b // blk,),
        in_specs=[
            pl.BlockSpec((blk, k), lambda i: (i, 0)),
            pl.BlockSpec((h, k), lambda i: (0, 0)),
            pl.BlockSpec((1, h), lambda i: (0, 0)),
            pl.BlockSpec((o, h), lambda i: (0, 0)),
        ],
        out_specs=pl.BlockSpec((blk, o), lambda i: (i, 0)),
        out_shape=jax.ShapeDtypeStruct((b, o), jnp.float32),
    )(win, W1, b1.reshape(1, h), W2)


def kernel(x, table, W1, b1, W2):
    batch, win = x.shape
    d = table.shape[1]
    xi = x.astype(jnp.int32).reshape(_NW, _NCHUNKS, _CHUNK)
    q3d = xi >> 2
    sb3d = (xi & 3) * d
    table4 = table.reshape(table.shape[0] // 4, 4 * d)
    packed = _sc_gather_extract(table4, q3d, sb3d)
    win_emb = packed.reshape(batch, win * d)
    return _mlp(win_emb, W1, b1, W2)


# speedup vs baseline: 4.1585x; 2.2858x over previous
import functools

import jax
import jax.numpy as jnp
from jax import lax
from jax.experimental import pallas as pl
from jax.experimental.pallas import tpu as pltpu
from jax.experimental.pallas import tpu_sc as plsc

_NUM_CORES = 2
_NUM_SUBCORES = 16
_NW = _NUM_CORES * _NUM_SUBCORES
_CHUNK = 128
_NCHUNKS = 20
_BLK = 128
_PITCH = 64
_D = 50
_SPG = 4
_VB = 2048


def _tc_pack(tt, n_out_rows):
    d, v = tt.shape
    grid = (n_out_rows * 2 + _VB - 1) // _VB
    half = _VB // 2

    def body(in_ref, out_ref):
        x = in_ref[...]
        xt = x.T
        xp = jnp.pad(xt, ((0, 0), (0, _PITCH - d)))
        out_ref[...] = jnp.concatenate(
            [xp[:half], xp[half:]], axis=1)

    return pl.pallas_call(
        body,
        grid=(grid,),
        in_specs=[pl.BlockSpec((d, _VB), lambda i: (0, i))],
        out_specs=pl.BlockSpec((_VB // 2, 128), lambda i: (i, 0)),
        out_shape=jax.ShapeDtypeStruct((n_out_rows, 128), jnp.float32),
    )(tt)


def _sc_gather_extract(table4, q3d, sb3d):
    n_rows_out = _NW * _NCHUNKS * _CHUNK * _D // 128
    mesh = plsc.VectorSubcoreMesh(core_axis_name="c", subcore_axis_name="s")
    stage_words = _SPG * _CHUNK * _D

    @functools.partial(
        pl.kernel,
        out_type=jax.ShapeDtypeStruct((n_rows_out, 128), jnp.float32),
        mesh=mesh,
        scratch_types=[
            pltpu.VMEM((_NCHUNKS, _CHUNK), jnp.int32),
            pltpu.VMEM((_NCHUNKS, _CHUNK), jnp.int32),
            pltpu.VMEM((_CHUNK, _BLK), jnp.float32),
            pltpu.VMEM((stage_words // 128, 128), jnp.float32),
            pltpu.SemaphoreType.DMA,
        ],
        compiler_params=pltpu.CompilerParams(
            use_tc_tiling_on_sc=False, needs_layout_passes=False
        ),
    )
    def gather_kernel(table_hbm, q_hbm, sb_hbm, out_hbm, q_v, sb_v, rows_v,
                      stage_v, sem):
        wid = lax.axis_index("s") * _NUM_CORES + lax.axis_index("c")
        pltpu.sync_copy(q_hbm.at[wid], q_v)
        pltpu.sync_copy(sb_hbm.at[wid], sb_v)
        lane = lax.iota(jnp.int32, 16)
        lane50 = lane * _D

        def chunk_body(ch, carry):
            pltpu.async_copy(table_hbm.at[q_v.at[ch]], rows_v, sem).wait()
            st_base = (ch % _SPG) * (_CHUNK * _D)
            for j in range(_CHUNK // 16):
                row_vec = lane + 16 * j
                col0 = sb_v[ch, pl.ds(16 * j, 16)]
                dst0 = st_base + (16 * j) * _D + lane50
                for k in range(_D):
                    v = plsc.load_gather(rows_v, [row_vec, col0 + k])
                    dst = dst0 + k
                    plsc.store_scatter(stage_v, [dst >> 7, dst & 127], v)
            @pl.when(ch % _SPG == _SPG - 1)
            def _():
                g = ch // _SPG
                off = wid * (_NCHUNKS * _CHUNK * _D // 128) + g * (stage_words // 128)
                pltpu.sync_copy(
                    stage_v, out_hbm.at[pl.ds(off, stage_words // 128)]
                )
            return carry

        lax.fori_loop(0, _NCHUNKS, chunk_body, None)

    return gather_kernel(table4, q3d, sb3d)


def _mlp(win, W1, b1, W2):
    b, k = win.shape
    h = W1.shape[0]
    o = W2.shape[0]
    blk = 2048

    def body(win_ref, w1_ref, b1_ref, w2_ref, out_ref):
        z = lax.dot_general(
            win_ref[...], w1_ref[...],
            (((1,), (1,)), ((), ())),
            preferred_element_type=jnp.float32,
        )
        act = jax.nn.sigmoid(z + b1_ref[...])
        out_ref[...] = lax.dot_general(
            act, w2_ref[...],
            (((1,), (1,)), ((), ())),
            preferred_element_type=jnp.float32,
        )

    return pl.pallas_call(
        body,
        grid=(b // blk,),
        in_specs=[
            pl.BlockSpec((blk, k), lambda i: (i, 0)),
            pl.BlockSpec((h, k), lambda i: (0, 0)),
            pl.BlockSpec((1, h), lambda i: (0, 0)),
            pl.BlockSpec((o, h), lambda i: (0, 0)),
        ],
        out_specs=pl.BlockSpec((blk, o), lambda i: (i, 0)),
        out_shape=jax.ShapeDtypeStruct((b, o), jnp.float32),
    )(win, W1, b1.reshape(1, h), W2)


def kernel(x, table, W1, b1, W2):
    batch, win = x.shape
    v, d = table.shape
    xi = x.astype(jnp.int32).reshape(_NW, _NCHUNKS, _CHUNK)
    q3d = ((xi >> 11) << 10) + (xi & 1023)
    sb3d = ((xi >> 10) & 1) * _PITCH
    grid = (v + _VB - 1) // _VB
    table2 = _tc_pack(table.T, grid * (_VB // 2))
    packed = _sc_gather_extract(table2, q3d, sb3d)
    win_emb = packed.reshape(batch, win * d)
    return _mlp(win_emb, W1, b1, W2)


# speedup vs baseline: 5.7787x; 1.3896x over previous
import functools

import jax
import jax.numpy as jnp
from jax import lax
from jax.experimental import pallas as pl
from jax.experimental.pallas import tpu as pltpu
from jax.experimental.pallas import tpu_sc as plsc

_NUM_CORES = 2
_NUM_SUBCORES = 16
_NW = _NUM_CORES * _NUM_SUBCORES
_CHUNK = 128
_NCHUNKS = 20
_BLK = 128
_PITCH = 64
_D = 50
_SPG = 4
_VB = 8192


def _tc_pack(tt, n_out_rows):
    d, v = tt.shape
    grid = (n_out_rows * 2 + _VB - 1) // _VB
    half = _VB // 2

    def body(in_ref, out_ref):
        x = in_ref[...]
        xt = x.T
        xp = jnp.pad(xt, ((0, 0), (0, _PITCH - d)))
        out_ref[...] = jnp.concatenate(
            [xp[:half], xp[half:]], axis=1)

    return pl.pallas_call(
        body,
        grid=(grid,),
        in_specs=[pl.BlockSpec((d, _VB), lambda i: (0, i))],
        out_specs=pl.BlockSpec((_VB // 2, 128), lambda i: (i, 0)),
        out_shape=jax.ShapeDtypeStruct((n_out_rows, 128), jnp.float32),
    )(tt)


def _sc_gather_extract(table4, q3d, sb3d):
    n_rows_out = _NW * _NCHUNKS * _CHUNK * _D // 128
    mesh = plsc.VectorSubcoreMesh(core_axis_name="c", subcore_axis_name="s")
    stage_words = _SPG * _CHUNK * _D

    @functools.partial(
        pl.kernel,
        out_type=jax.ShapeDtypeStruct((n_rows_out, 128), jnp.float32),
        mesh=mesh,
        scratch_types=[
            pltpu.VMEM((_NCHUNKS, _CHUNK), jnp.int32),
            pltpu.VMEM((_NCHUNKS, _CHUNK), jnp.int32),
            pltpu.VMEM((_CHUNK, _BLK), jnp.float32),
            pltpu.VMEM((stage_words // 128, 128), jnp.float32),
            pltpu.SemaphoreType.DMA,
        ],
        compiler_params=pltpu.CompilerParams(
            use_tc_tiling_on_sc=False, needs_layout_passes=False
        ),
    )
    def gather_kernel(table_hbm, q_hbm, sb_hbm, out_hbm, q_v, sb_v, rows_v,
                      stage_v, sem):
        wid = lax.axis_index("s") * _NUM_CORES + lax.axis_index("c")
        pltpu.sync_copy(q_hbm.at[wid], q_v)
        pltpu.sync_copy(sb_hbm.at[wid], sb_v)
        lane = lax.iota(jnp.int32, 16)
        lane50 = lane * _D

        def chunk_body(ch, carry):
            pltpu.async_copy(table_hbm.at[q_v.at[ch]], rows_v, sem).wait()
            st_base = (ch % _SPG) * (_CHUNK * _D)
            for j in range(_CHUNK // 16):
                row_vec = lane + 16 * j
                col0 = sb_v[ch, pl.ds(16 * j, 16)]
                dst0 = st_base + (16 * j) * _D + lane50
                for k in range(_D):
                    v = plsc.load_gather(rows_v, [row_vec, col0 + k])
                    dst = dst0 + k
                    plsc.store_scatter(stage_v, [dst >> 7, dst & 127], v)
            @pl.when(ch % _SPG == _SPG - 1)
            def _():
                g = ch // _SPG
                off = wid * (_NCHUNKS * _CHUNK * _D // 128) + g * (stage_words // 128)
                pltpu.sync_copy(
                    stage_v, out_hbm.at[pl.ds(off, stage_words // 128)]
                )
            return carry

        lax.fori_loop(0, _NCHUNKS, chunk_body, None)

    return gather_kernel(table4, q3d, sb3d)


def _mlp(win, W1, b1, W2):
    b, k = win.shape
    h = W1.shape[0]
    o = W2.shape[0]
    blk = 2048

    def body(win_ref, w1_ref, b1_ref, w2_ref, out_ref):
        z = lax.dot_general(
            win_ref[...], w1_ref[...],
            (((1,), (1,)), ((), ())),
            preferred_element_type=jnp.float32,
        )
        act = jax.nn.sigmoid(z + b1_ref[...])
        out_ref[...] = lax.dot_general(
            act, w2_ref[...],
            (((1,), (1,)), ((), ())),
            preferred_element_type=jnp.float32,
        )

    return pl.pallas_call(
        body,
        grid=(b // blk,),
        in_specs=[
            pl.BlockSpec((blk, k), lambda i: (i, 0)),
            pl.BlockSpec((h, k), lambda i: (0, 0)),
            pl.BlockSpec((1, h), lambda i: (0, 0)),
            pl.BlockSpec((o, h), lambda i: (0, 0)),
        ],
        out_specs=pl.BlockSpec((blk, o), lambda i: (i, 0)),
        out_shape=jax.ShapeDtypeStruct((b, o), jnp.float32),
    )(win, W1, b1.reshape(1, h), W2)


def kernel(x, table, W1, b1, W2):
    batch, win = x.shape
    v, d = table.shape
    xi = x.astype(jnp.int32).reshape(_NW, _NCHUNKS, _CHUNK)
    hb = _VB.bit_length() - 2
    q3d = ((xi >> (hb + 1)) << hb) + (xi & ((1 << hb) - 1))
    sb3d = ((xi >> hb) & 1) * _PITCH
    grid = (v + _VB - 1) // _VB
    table2 = _tc_pack(table.T, grid * (_VB // 2))
    packed = _sc_gather_extract(table2, q3d, sb3d)
    win_emb = packed.reshape(batch, win * d)
    return _mlp(win_emb, W1, b1, W2)


# speedup vs baseline: 6.1822x; 1.0698x over previous
import functools

import jax
import jax.numpy as jnp
from jax import lax
from jax.experimental import pallas as pl
from jax.experimental.pallas import tpu as pltpu
from jax.experimental.pallas import tpu_sc as plsc

_NUM_CORES = 2
_NUM_SUBCORES = 16
_NW = _NUM_CORES * _NUM_SUBCORES
_CHUNK = 128
_NCHUNKS = 20
_BLK = 128
_PITCH = 64
_D = 50
_SPG = 4
_VB = 16384


def _tc_pack(tt, n_out_rows):
    d, v = tt.shape
    grid = (n_out_rows * 2 + _VB - 1) // _VB
    half = _VB // 2

    def body(in_ref, out_ref):
        x = in_ref[...]
        xt = x.T
        xp = jnp.pad(xt, ((0, 0), (0, _PITCH - d)))
        out_ref[...] = jnp.concatenate(
            [xp[:half], xp[half:]], axis=1)

    return pl.pallas_call(
        body,
        grid=(grid,),
        in_specs=[pl.BlockSpec((d, _VB), lambda i: (0, i))],
        out_specs=pl.BlockSpec((_VB // 2, 128), lambda i: (i, 0)),
        out_shape=jax.ShapeDtypeStruct((n_out_rows, 128), jnp.float32),
    )(tt)


def _sc_gather_extract(table4, q3d, sb3d):
    n_rows_out = _NW * _NCHUNKS * _CHUNK * _D // 128
    mesh = plsc.VectorSubcoreMesh(core_axis_name="c", subcore_axis_name="s")
    stage_words = _SPG * _CHUNK * _D

    @functools.partial(
        pl.kernel,
        out_type=jax.ShapeDtypeStruct((n_rows_out, 128), jnp.float32),
        mesh=mesh,
        scratch_types=[
            pltpu.VMEM((_NCHUNKS, _CHUNK), jnp.int32),
            pltpu.VMEM((_NCHUNKS, _CHUNK), jnp.int32),
            pltpu.VMEM((_CHUNK, _BLK), jnp.float32),
            pltpu.VMEM((stage_words // 128, 128), jnp.float32),
            pltpu.SemaphoreType.DMA,
        ],
        compiler_params=pltpu.CompilerParams(
            use_tc_tiling_on_sc=False, needs_layout_passes=False
        ),
    )
    def gather_kernel(table_hbm, q_hbm, sb_hbm, out_hbm, q_v, sb_v, rows_v,
                      stage_v, sem):
        wid = lax.axis_index("s") * _NUM_CORES + lax.axis_index("c")
        pltpu.sync_copy(q_hbm.at[wid], q_v)
        pltpu.sync_copy(sb_hbm.at[wid], sb_v)
        lane = lax.iota(jnp.int32, 16)
        lane50 = lane * _D

        def chunk_body(ch, carry):
            pltpu.async_copy(table_hbm.at[q_v.at[ch]], rows_v, sem).wait()
            st_base = (ch % _SPG) * (_CHUNK * _D)
            for j in range(_CHUNK // 16):
                row_vec = lane + 16 * j
                col0 = sb_v[ch, pl.ds(16 * j, 16)]
                dst0 = st_base + (16 * j) * _D + lane50
                for k in range(_D):
                    v = plsc.load_gather(rows_v, [row_vec, col0 + k])
                    dst = dst0 + k
                    plsc.store_scatter(stage_v, [dst >> 7, dst & 127], v)
            @pl.when(ch % _SPG == _SPG - 1)
            def _():
                g = ch // _SPG
                off = wid * (_NCHUNKS * _CHUNK * _D // 128) + g * (stage_words // 128)
                pltpu.sync_copy(
                    stage_v, out_hbm.at[pl.ds(off, stage_words // 128)]
                )
            return carry

        lax.fori_loop(0, _NCHUNKS, chunk_body, None)

    return gather_kernel(table4, q3d, sb3d)


def _mlp(win, W1, b1, W2):
    b, k = win.shape
    h = W1.shape[0]
    o = W2.shape[0]
    blk = 2048

    def body(win_ref, w1_ref, b1_ref, w2_ref, out_ref):
        z = lax.dot_general(
            win_ref[...], w1_ref[...],
            (((1,), (1,)), ((), ())),
            preferred_element_type=jnp.float32,
        )
        act = jax.nn.sigmoid(z + b1_ref[...])
        out_ref[...] = lax.dot_general(
            act, w2_ref[...],
            (((1,), (1,)), ((), ())),
            preferred_element_type=jnp.float32,
        )

    return pl.pallas_call(
        body,
        grid=(b // blk,),
        in_specs=[
            pl.BlockSpec((blk, k), lambda i: (i, 0)),
            pl.BlockSpec((h, k), lambda i: (0, 0)),
            pl.BlockSpec((1, h), lambda i: (0, 0)),
            pl.BlockSpec((o, h), lambda i: (0, 0)),
        ],
        out_specs=pl.BlockSpec((blk, o), lambda i: (i, 0)),
        out_shape=jax.ShapeDtypeStruct((b, o), jnp.float32),
    )(win, W1, b1.reshape(1, h), W2)


def kernel(x, table, W1, b1, W2):
    batch, win = x.shape
    v, d = table.shape
    xi = x.astype(jnp.int32).reshape(_NW, _NCHUNKS, _CHUNK)
    hb = _VB.bit_length() - 2
    q3d = ((xi >> (hb + 1)) << hb) + (xi & ((1 << hb) - 1))
    sb3d = ((xi >> hb) & 1) * _PITCH
    grid = (v + _VB - 1) // _VB
    table2 = _tc_pack(table.T, grid * (_VB // 2))
    packed = _sc_gather_extract(table2, q3d, sb3d)
    win_emb = packed.reshape(batch, win * d)
    return _mlp(win_emb, W1, b1, W2)


# speedup vs baseline: 6.3115x; 1.0209x over previous
import functools

import jax
import jax.numpy as jnp
from jax import lax
from jax.experimental import pallas as pl
from jax.experimental.pallas import tpu as pltpu
from jax.experimental.pallas import tpu_sc as plsc

_NUM_CORES = 2
_NUM_SUBCORES = 16
_NW = _NUM_CORES * _NUM_SUBCORES
_CHUNK = 128
_NCHUNKS = 20
_BLK = 128
_PITCH = 64
_D = 50
_SPG = 4
_VB = 32768


def _tc_pack(tt, n_out_rows):
    d, v = tt.shape
    grid = (n_out_rows * 2 + _VB - 1) // _VB
    half = _VB // 2

    def body(in_ref, out_ref):
        x = in_ref[...]
        xt = x.T
        xp = jnp.pad(xt, ((0, 0), (0, _PITCH - d)))
        out_ref[...] = jnp.concatenate(
            [xp[:half], xp[half:]], axis=1)

    return pl.pallas_call(
        body,
        grid=(grid,),
        in_specs=[pl.BlockSpec((d, _VB), lambda i: (0, i))],
        out_specs=pl.BlockSpec((_VB // 2, 128), lambda i: (i, 0)),
        out_shape=jax.ShapeDtypeStruct((n_out_rows, 128), jnp.float32),
    )(tt)


def _sc_gather_extract(table4, q3d, sb3d):
    n_rows_out = _NW * _NCHUNKS * _CHUNK * _D // 128
    mesh = plsc.VectorSubcoreMesh(core_axis_name="c", subcore_axis_name="s")
    stage_words = _SPG * _CHUNK * _D

    @functools.partial(
        pl.kernel,
        out_type=jax.ShapeDtypeStruct((n_rows_out, 128), jnp.float32),
        mesh=mesh,
        scratch_types=[
            pltpu.VMEM((_NCHUNKS, _CHUNK), jnp.int32),
            pltpu.VMEM((_NCHUNKS, _CHUNK), jnp.int32),
            pltpu.VMEM((_CHUNK, _BLK), jnp.float32),
            pltpu.VMEM((stage_words // 128, 128), jnp.float32),
            pltpu.SemaphoreType.DMA,
        ],
        compiler_params=pltpu.CompilerParams(
            use_tc_tiling_on_sc=False, needs_layout_passes=False
        ),
    )
    def gather_kernel(table_hbm, q_hbm, sb_hbm, out_hbm, q_v, sb_v, rows_v,
                      stage_v, sem):
        wid = lax.axis_index("s") * _NUM_CORES + lax.axis_index("c")
        pltpu.sync_copy(q_hbm.at[wid], q_v)
        pltpu.sync_copy(sb_hbm.at[wid], sb_v)
        lane = lax.iota(jnp.int32, 16)
        lane50 = lane * _D

        def chunk_body(ch, carry):
            pltpu.async_copy(table_hbm.at[q_v.at[ch]], rows_v, sem).wait()
            st_base = (ch % _SPG) * (_CHUNK * _D)
            for j in range(_CHUNK // 16):
                row_vec = lane + 16 * j
                col0 = sb_v[ch, pl.ds(16 * j, 16)]
                dst0 = st_base + (16 * j) * _D + lane50
                for k in range(_D):
                    v = plsc.load_gather(rows_v, [row_vec, col0 + k])
                    dst = dst0 + k
                    plsc.store_scatter(stage_v, [dst >> 7, dst & 127], v)
            @pl.when(ch % _SPG == _SPG - 1)
            def _():
                g = ch // _SPG
                off = wid * (_NCHUNKS * _CHUNK * _D // 128) + g * (stage_words // 128)
                pltpu.sync_copy(
                    stage_v, out_hbm.at[pl.ds(off, stage_words // 128)]
                )
            return carry

        lax.fori_loop(0, _NCHUNKS, chunk_body, None)

    return gather_kernel(table4, q3d, sb3d)


def _mlp(win, W1, b1, W2):
    b, k = win.shape
    h = W1.shape[0]
    o = W2.shape[0]
    blk = 2048

    def body(win_ref, w1_ref, b1_ref, w2_ref, out_ref):
        z = lax.dot_general(
            win_ref[...], w1_ref[...],
            (((1,), (1,)), ((), ())),
            preferred_element_type=jnp.float32,
        )
        act = jax.nn.sigmoid(z + b1_ref[...])
        out_ref[...] = lax.dot_general(
            act, w2_ref[...],
            (((1,), (1,)), ((), ())),
            preferred_element_type=jnp.float32,
        )

    return pl.pallas_call(
        body,
        grid=(b // blk,),
        in_specs=[
            pl.BlockSpec((blk, k), lambda i: (i, 0)),
            pl.BlockSpec((h, k), lambda i: (0, 0)),
            pl.BlockSpec((1, h), lambda i: (0, 0)),
            pl.BlockSpec((o, h), lambda i: (0, 0)),
        ],
        out_specs=pl.BlockSpec((blk, o), lambda i: (i, 0)),
        out_shape=jax.ShapeDtypeStruct((b, o), jnp.float32),
    )(win, W1, b1.reshape(1, h), W2)


def kernel(x, table, W1, b1, W2):
    batch, win = x.shape
    v, d = table.shape
    xi = x.astype(jnp.int32).reshape(_NW, _NCHUNKS, _CHUNK)
    hb = _VB.bit_length() - 2
    q3d = ((xi >> (hb + 1)) << hb) + (xi & ((1 << hb) - 1))
    sb3d = ((xi >> hb) & 1) * _PITCH
    grid = (v + _VB - 1) // _VB
    table2 = _tc_pack(table.T, grid * (_VB // 2))
    packed = _sc_gather_extract(table2, q3d, sb3d)
    win_emb = packed.reshape(batch, win * d)
    return _mlp(win_emb, W1, b1, W2)


# speedup vs baseline: 6.3711x; 1.0094x over previous
import functools

import jax
import jax.numpy as jnp
from jax import lax
from jax.experimental import pallas as pl
from jax.experimental.pallas import tpu as pltpu
from jax.experimental.pallas import tpu_sc as plsc

_NUM_CORES = 2
_NUM_SUBCORES = 16
_NW = _NUM_CORES * _NUM_SUBCORES
_CHUNK = 128
_NCHUNKS = 20
_BLK = 128
_PITCH = 64
_D = 50
_SPG = 4
_VB = 32768


def _tc_pack(tt, n_out_rows):
    d, v = tt.shape
    grid = (n_out_rows * 2 + _VB - 1) // _VB
    half = _VB // 2

    def body(in_ref, out_ref):
        x = in_ref[...]
        xt = x.T
        xp = jnp.pad(xt, ((0, 0), (0, _PITCH - d)))
        out_ref[...] = jnp.concatenate(
            [xp[:half], xp[half:]], axis=1)

    return pl.pallas_call(
        body,
        grid=(grid,),
        in_specs=[pl.BlockSpec((d, _VB), lambda i: (0, i))],
        out_specs=pl.BlockSpec((_VB // 2, 128), lambda i: (i, 0)),
        out_shape=jax.ShapeDtypeStruct((n_out_rows, 128), jnp.float32),
    )(tt)


def _sc_gather_extract(table4, q3d, sb3d):
    n_rows_out = _NW * _NCHUNKS * _CHUNK * _D // 128
    mesh = plsc.VectorSubcoreMesh(core_axis_name="c", subcore_axis_name="s")
    stage_words = _SPG * _CHUNK * _D

    @functools.partial(
        pl.kernel,
        out_type=jax.ShapeDtypeStruct((n_rows_out, 128), jnp.float32),
        mesh=mesh,
        scratch_types=[
            pltpu.VMEM((_NCHUNKS, _CHUNK), jnp.int32),
            pltpu.VMEM((_NCHUNKS, _CHUNK), jnp.int32),
            pltpu.VMEM((_SPG, _CHUNK, _BLK), jnp.float32),
            pltpu.VMEM((stage_words // 128, 128), jnp.float32),
            pltpu.SemaphoreType.DMA,
            pltpu.SemaphoreType.DMA,
            pltpu.SemaphoreType.DMA,
            pltpu.SemaphoreType.DMA,
        ],
        compiler_params=pltpu.CompilerParams(
            use_tc_tiling_on_sc=False, needs_layout_passes=False
        ),
    )
    def gather_kernel(table_hbm, q_hbm, sb_hbm, out_hbm, q_v, sb_v, rows_v,
                      stage_v, sem0, sem1, sem2, sem3):
        wid = lax.axis_index("s") * _NUM_CORES + lax.axis_index("c")
        sems = (sem0, sem1, sem2, sem3)
        pltpu.sync_copy(q_hbm.at[wid], q_v)
        pltpu.sync_copy(sb_hbm.at[wid], sb_v)
        lane = lax.iota(jnp.int32, 16)
        lane50 = lane * _D

        def group_body(g, carry):
            copies = [
                pltpu.async_copy(
                    table_hbm.at[q_v.at[_SPG * g + b]], rows_v.at[b], sems[b]
                )
                for b in range(_SPG)
            ]
            for b in range(_SPG):
                copies[b].wait()
                st_base = b * (_CHUNK * _D)
                for j in range(_CHUNK // 16):
                    row_vec = lane + 16 * j
                    col0 = sb_v[_SPG * g + b, pl.ds(16 * j, 16)]
                    dst0 = st_base + (16 * j) * _D + lane50
                    for k in range(_D):
                        v = plsc.load_gather(rows_v.at[b], [row_vec, col0 + k])
                        dst = dst0 + k
                        plsc.store_scatter(stage_v, [dst >> 7, dst & 127], v)
            off = wid * (_NCHUNKS * _CHUNK * _D // 128) + g * (stage_words // 128)
            pltpu.sync_copy(stage_v, out_hbm.at[pl.ds(off, stage_words // 128)])
            return carry

        lax.fori_loop(0, _NCHUNKS // _SPG, group_body, None)

    return gather_kernel(table4, q3d, sb3d)


def _mlp(win, W1, b1, W2):
    b, k = win.shape
    h = W1.shape[0]
    o = W2.shape[0]
    blk = 2048

    def body(win_ref, w1_ref, b1_ref, w2_ref, out_ref):
        z = lax.dot_general(
            win_ref[...], w1_ref[...],
            (((1,), (1,)), ((), ())),
            preferred_element_type=jnp.float32,
        )
        act = jax.nn.sigmoid(z + b1_ref[...])
        out_ref[...] = lax.dot_general(
            act, w2_ref[...],
            (((1,), (1,)), ((), ())),
            preferred_element_type=jnp.float32,
        )

    return pl.pallas_call(
        body,
        grid=(b // blk,),
        in_specs=[
            pl.BlockSpec((blk, k), lambda i: (i, 0)),
            pl.BlockSpec((h, k), lambda i: (0, 0)),
            pl.BlockSpec((1, h), lambda i: (0, 0)),
            pl.BlockSpec((o, h), lambda i: (0, 0)),
        ],
        out_specs=pl.BlockSpec((blk, o), lambda i: (i, 0)),
        out_shape=jax.ShapeDtypeStruct((b, o), jnp.float32),
    )(win, W1, b1.reshape(1, h), W2)


def kernel(x, table, W1, b1, W2):
    batch, win = x.shape
    v, d = table.shape
    xi = x.astype(jnp.int32).reshape(_NW, _NCHUNKS, _CHUNK)
    hb = _VB.bit_length() - 2
    q3d = ((xi >> (hb + 1)) << hb) + (xi & ((1 << hb) - 1))
    sb3d = ((xi >> hb) & 1) * _PITCH
    grid = (v + _VB - 1) // _VB
    table2 = _tc_pack(table.T, grid * (_VB // 2))
    packed = _sc_gather_extract(table2, q3d, sb3d)
    win_emb = packed.reshape(batch, win * d)
    return _mlp(win_emb, W1, b1, W2)


# speedup vs baseline: 7.2310x; 1.1350x over previous
import functools

import jax
import jax.numpy as jnp
from jax import lax
from jax.experimental import pallas as pl
from jax.experimental.pallas import tpu as pltpu
from jax.experimental.pallas import tpu_sc as plsc

_NUM_CORES = 2
_NUM_SUBCORES = 16
_NW = _NUM_CORES * _NUM_SUBCORES
_CHUNK = 128
_NCHUNKS = 20
_BLK = 128
_PITCH = 64
_D = 50
_SPG = 4
_VB = 32768


def _tc_pack(tt, n_out_rows):
    d, v = tt.shape
    grid = (n_out_rows * 2 + _VB - 1) // _VB
    half = _VB // 2

    def body(in_ref, out_ref):
        x = in_ref[...]
        xt = x.T
        xp = jnp.pad(xt, ((0, 0), (0, _PITCH - d)))
        out_ref[...] = jnp.concatenate(
            [xp[:half], xp[half:]], axis=1)

    return pl.pallas_call(
        body,
        grid=(grid,),
        in_specs=[pl.BlockSpec((d, _VB), lambda i: (0, i))],
        out_specs=pl.BlockSpec((_VB // 2, 128), lambda i: (i, 0)),
        out_shape=jax.ShapeDtypeStruct((n_out_rows, 128), jnp.float32),
    )(tt)


def _sc_gather_extract(table4, q3d, sb3d):
    n_rows_out = _NW * _NCHUNKS * _CHUNK * _D // 128
    mesh = plsc.VectorSubcoreMesh(core_axis_name="c", subcore_axis_name="s")
    stage_words = _SPG * _CHUNK * _D

    @functools.partial(
        pl.kernel,
        out_type=jax.ShapeDtypeStruct((n_rows_out, 128), jnp.float32),
        mesh=mesh,
        scratch_types=[
            pltpu.VMEM((_NCHUNKS, _CHUNK), jnp.int32),
            pltpu.VMEM((_NCHUNKS, _CHUNK), jnp.int32),
            pltpu.VMEM((_SPG, _CHUNK, _BLK), jnp.float32),
            pltpu.VMEM((stage_words // 128, 128), jnp.float32),
            pltpu.SemaphoreType.DMA,
            pltpu.SemaphoreType.DMA,
            pltpu.SemaphoreType.DMA,
            pltpu.SemaphoreType.DMA,
        ],
        compiler_params=pltpu.CompilerParams(
            use_tc_tiling_on_sc=False, needs_layout_passes=False
        ),
    )
    def gather_kernel(table_hbm, q_hbm, sb_hbm, out_hbm, q_v, sb_v, rows_v,
                      stage_v, sem0, sem1, sem2, sem3):
        wid = lax.axis_index("s") * _NUM_CORES + lax.axis_index("c")
        sems = (sem0, sem1, sem2, sem3)
        pltpu.sync_copy(q_hbm.at[wid], q_v)
        pltpu.sync_copy(sb_hbm.at[wid], sb_v)
        lane = lax.iota(jnp.int32, 16)
        lane50 = lane * _D

        def group_body(g, carry):
            copies = [
                pltpu.async_copy(
                    table_hbm.at[q_v.at[_SPG * g + b]],
                    rows_v.at[b],
                    sems[b],
                )
                for b in range(_SPG)
            ]
            for b in range(_SPG):
                copies[b].wait()
                st_base = b * (_CHUNK * _D)

                def row_body(j, carry2, b=b):
                    jv = jnp.full((16,), j, jnp.int32)
                    sbv = plsc.load_gather(
                        sb_v, [jnp.full((16,), _SPG * g + b, jnp.int32), jv])
                    dstb = st_base + j * _D
                    for t in (0, 16, 32, _D - 16):
                        v = plsc.load_gather(
                            rows_v.at[b], [jv, sbv + (lane + t)])
                        dst = dstb + (lane + t)
                        plsc.store_scatter(stage_v, [dst >> 7, dst & 127], v)
                    return carry2

                lax.fori_loop(0, _CHUNK, row_body, None, unroll=16)
            off = wid * (_NCHUNKS * _CHUNK * _D // 128) + g * (stage_words // 128)
            pltpu.sync_copy(stage_v, out_hbm.at[pl.ds(off, stage_words // 128)])
            return carry

        lax.fori_loop(0, _NCHUNKS // _SPG, group_body, None)

    return gather_kernel(table4, q3d, sb3d)


def _mlp(win, W1, b1, W2):
    b, k = win.shape
    h = W1.shape[0]
    o = W2.shape[0]
    blk = 2048

    def body(win_ref, w1_ref, b1_ref, w2_ref, out_ref):
        z = lax.dot_general(
            win_ref[...], w1_ref[...],
            (((1,), (1,)), ((), ())),
            preferred_element_type=jnp.float32,
        )
        act = jax.nn.sigmoid(z + b1_ref[...])
        out_ref[...] = lax.dot_general(
            act, w2_ref[...],
            (((1,), (1,)), ((), ())),
            preferred_element_type=jnp.float32,
        )

    return pl.pallas_call(
        body,
        grid=(b // blk,),
        in_specs=[
            pl.BlockSpec((blk, k), lambda i: (i, 0)),
            pl.BlockSpec((h, k), lambda i: (0, 0)),
            pl.BlockSpec((1, h), lambda i: (0, 0)),
            pl.BlockSpec((o, h), lambda i: (0, 0)),
        ],
        out_specs=pl.BlockSpec((blk, o), lambda i: (i, 0)),
        out_shape=jax.ShapeDtypeStruct((b, o), jnp.float32),
    )(win, W1, b1.reshape(1, h), W2)


def kernel(x, table, W1, b1, W2):
    batch, win = x.shape
    v, d = table.shape
    xi = x.astype(jnp.int32).reshape(_NW, _NCHUNKS, _CHUNK)
    hb = _VB.bit_length() - 2
    q3d = ((xi >> (hb + 1)) << hb) + (xi & ((1 << hb) - 1))
    sb3d = ((xi >> hb) & 1) * _PITCH
    grid = (v + _VB - 1) // _VB
    table2 = _tc_pack(table.T, grid * (_VB // 2))
    packed = _sc_gather_extract(table2, q3d, sb3d)
    win_emb = packed.reshape(batch, win * d)
    return _mlp(win_emb, W1, b1, W2)


# speedup vs baseline: 8.6551x; 1.1969x over previous
import functools

import jax
import jax.numpy as jnp
from jax import lax
from jax.experimental import pallas as pl
from jax.experimental.pallas import tpu as pltpu
from jax.experimental.pallas import tpu_sc as plsc

_NUM_CORES = 2
_NUM_SUBCORES = 16
_NW = _NUM_CORES * _NUM_SUBCORES
_CHUNK = 128
_NCHUNKS = 20
_BLK = 128
_PITCH = 64
_D = 50
_SPG = 4
_VB = 32768


def _tc_pack(tt, n_out_rows):
    d, v = tt.shape
    grid = (n_out_rows * 2 + _VB - 1) // _VB
    half = _VB // 2

    def body(in_ref, out_ref):
        x = in_ref[...]
        xt = x.T
        xp = jnp.pad(xt, ((0, 0), (0, _PITCH - d)))
        out_ref[...] = jnp.concatenate(
            [xp[:half], xp[half:]], axis=1)

    return pl.pallas_call(
        body,
        grid=(grid,),
        in_specs=[pl.BlockSpec((d, _VB), lambda i: (0, i))],
        out_specs=pl.BlockSpec((_VB // 2, 128), lambda i: (i, 0)),
        out_shape=jax.ShapeDtypeStruct((n_out_rows, 128), jnp.float32),
    )(tt)


def _sc_gather_extract(table64, p3d):
    n_out = _NW * _NCHUNKS * _CHUNK * _D
    mesh = plsc.VectorSubcoreMesh(core_axis_name="c", subcore_axis_name="s")
    stage_words = _SPG * _CHUNK * _D

    @functools.partial(
        pl.kernel,
        out_type=jax.ShapeDtypeStruct((n_out,), jnp.float32),
        mesh=mesh,
        scratch_types=[
            pltpu.VMEM((_NCHUNKS, _CHUNK), jnp.int32),
            pltpu.VMEM((_SPG, _CHUNK, _PITCH), jnp.float32),
            pltpu.VMEM((stage_words,), jnp.float32),
            pltpu.SemaphoreType.DMA,
            pltpu.SemaphoreType.DMA,
            pltpu.SemaphoreType.DMA,
            pltpu.SemaphoreType.DMA,
        ],
        compiler_params=pltpu.CompilerParams(
            use_tc_tiling_on_sc=False, needs_layout_passes=False
        ),
    )
    def gather_kernel(table_hbm, p_hbm, out_hbm, p_v, rows_v, stage_v,
                      sem0, sem1, sem2, sem3):
        wid = lax.axis_index("s") * _NUM_CORES + lax.axis_index("c")
        sems = (sem0, sem1, sem2, sem3)
        pltpu.sync_copy(p_hbm.at[wid], p_v)

        def group_body(g, carry):
            copies = [
                pltpu.async_copy(
                    table_hbm.at[p_v.at[_SPG * g + b]],
                    rows_v.at[b],
                    sems[b],
                )
                for b in range(_SPG)
            ]
            for b in range(_SPG):
                copies[b].wait()
                st_base = b * (_CHUNK * _D)
                for j in range(_CHUNK):
                    for t in (0, 16, 32, _D - 16):
                        stage_v[pl.ds(st_base + j * _D + t, 16)] = (
                            rows_v[b, j, pl.ds(t, 16)]
                        )
            off = wid * (_NCHUNKS * _CHUNK * _D) + g * stage_words
            pltpu.sync_copy(stage_v, out_hbm.at[pl.ds(off, stage_words)])
            return carry

        lax.fori_loop(0, _NCHUNKS // _SPG, group_body, None)

    return gather_kernel(table64, p3d)


def _mlp(win, W1, b1, W2):
    b, k = win.shape
    h = W1.shape[0]
    o = W2.shape[0]
    blk = 2048

    def body(win_ref, w1_ref, b1_ref, w2_ref, out_ref):
        z = lax.dot_general(
            win_ref[...], w1_ref[...],
            (((1,), (1,)), ((), ())),
            preferred_element_type=jnp.float32,
        )
        act = jax.nn.sigmoid(z + b1_ref[...])
        out_ref[...] = lax.dot_general(
            act, w2_ref[...],
            (((1,), (1,)), ((), ())),
            preferred_element_type=jnp.float32,
        )

    return pl.pallas_call(
        body,
        grid=(b // blk,),
        in_specs=[
            pl.BlockSpec((blk, k), lambda i: (i, 0)),
            pl.BlockSpec((h, k), lambda i: (0, 0)),
            pl.BlockSpec((1, h), lambda i: (0, 0)),
            pl.BlockSpec((o, h), lambda i: (0, 0)),
        ],
        out_specs=pl.BlockSpec((blk, o), lambda i: (i, 0)),
        out_shape=jax.ShapeDtypeStruct((b, o), jnp.float32),
    )(win, W1, b1.reshape(1, h), W2)


def kernel(x, table, W1, b1, W2):
    batch, win = x.shape
    v, d = table.shape
    xi = x.astype(jnp.int32).reshape(_NW, _NCHUNKS, _CHUNK)
    hb = _VB.bit_length() - 2
    p3d = (((xi >> (hb + 1)) << (hb + 1))
           + ((xi & ((1 << hb) - 1)) << 1)
           + ((xi >> hb) & 1))
    grid = (v + _VB - 1) // _VB
    table2 = _tc_pack(table.T, grid * (_VB // 2))
    table64 = table2.reshape(-1, _PITCH)
    packed = _sc_gather_extract(table64, p3d)
    win_emb = packed.reshape(batch, win * d)
    return _mlp(win_emb, W1, b1, W2)


# speedup vs baseline: 8.6634x; 1.0010x over previous
import functools

import jax
import jax.numpy as jnp
from jax import lax
from jax.experimental import pallas as pl
from jax.experimental.pallas import tpu as pltpu
from jax.experimental.pallas import tpu_sc as plsc

_NUM_CORES = 2
_NUM_SUBCORES = 16
_NW = _NUM_CORES * _NUM_SUBCORES
_CHUNK = 128
_NCHUNKS = 20
_BLK = 128
_PITCH = 64
_D = 50
_SPG = 4
_VB = 32768


def _tc_pack(tt, n_out_rows):
    d, v = tt.shape
    grid = (n_out_rows * 2 + _VB - 1) // _VB
    half = _VB // 2

    def body(in_ref, out_ref):
        x = in_ref[...]
        xt = x.T
        out_ref[:, 0:d] = xt[:half]
        out_ref[:, _PITCH:_PITCH + d] = xt[half:]

    return pl.pallas_call(
        body,
        grid=(grid,),
        in_specs=[pl.BlockSpec((d, _VB), lambda i: (0, i))],
        out_specs=pl.BlockSpec((_VB // 2, 128), lambda i: (i, 0)),
        out_shape=jax.ShapeDtypeStruct((n_out_rows, 128), jnp.float32),
    )(tt)


def _sc_gather_extract(table64, p3d):
    n_out = _NW * _NCHUNKS * _CHUNK * _D
    mesh = plsc.VectorSubcoreMesh(core_axis_name="c", subcore_axis_name="s")
    stage_words = _SPG * _CHUNK * _D

    @functools.partial(
        pl.kernel,
        out_type=jax.ShapeDtypeStruct((n_out,), jnp.float32),
        mesh=mesh,
        scratch_types=[
            pltpu.VMEM((_NCHUNKS, _CHUNK), jnp.int32),
            pltpu.VMEM((_SPG, _CHUNK, _PITCH), jnp.float32),
            pltpu.VMEM((stage_words,), jnp.float32),
            pltpu.SemaphoreType.DMA,
            pltpu.SemaphoreType.DMA,
            pltpu.SemaphoreType.DMA,
            pltpu.SemaphoreType.DMA,
        ],
        compiler_params=pltpu.CompilerParams(
            use_tc_tiling_on_sc=False, needs_layout_passes=False
        ),
    )
    def gather_kernel(table_hbm, p_hbm, out_hbm, p_v, rows_v, stage_v,
                      sem0, sem1, sem2, sem3):
        wid = lax.axis_index("s") * _NUM_CORES + lax.axis_index("c")
        sems = (sem0, sem1, sem2, sem3)
        pltpu.sync_copy(p_hbm.at[wid], p_v)

        def group_body(g, carry):
            copies = [
                pltpu.async_copy(
                    table_hbm.at[p_v.at[_SPG * g + b]],
                    rows_v.at[b],
                    sems[b],
                )
                for b in range(_SPG)
            ]
            for b in range(_SPG):
                copies[b].wait()
                st_base = b * (_CHUNK * _D)
                for j in range(_CHUNK):
                    for t in (0, 16, 32, _D - 16):
                        stage_v[pl.ds(st_base + j * _D + t, 16)] = (
                            rows_v[b, j, pl.ds(t, 16)]
                        )
            off = wid * (_NCHUNKS * _CHUNK * _D) + g * stage_words
            pltpu.sync_copy(stage_v, out_hbm.at[pl.ds(off, stage_words)])
            return carry

        lax.fori_loop(0, _NCHUNKS // _SPG, group_body, None)

    return gather_kernel(table64, p3d)


def _mlp(win, W1, b1, W2):
    b, k = win.shape
    h = W1.shape[0]
    o = W2.shape[0]
    blk = 2048

    def body(win_ref, w1_ref, b1_ref, w2_ref, out_ref):
        z = lax.dot_general(
            win_ref[...], w1_ref[...],
            (((1,), (1,)), ((), ())),
            preferred_element_type=jnp.float32,
        )
        act = jax.nn.sigmoid(z + b1_ref[...])
        out_ref[...] = lax.dot_general(
            act, w2_ref[...],
            (((1,), (1,)), ((), ())),
            preferred_element_type=jnp.float32,
        )

    return pl.pallas_call(
        body,
        grid=(b // blk,),
        in_specs=[
            pl.BlockSpec((blk, k), lambda i: (i, 0)),
            pl.BlockSpec((h, k), lambda i: (0, 0)),
            pl.BlockSpec((1, h), lambda i: (0, 0)),
            pl.BlockSpec((o, h), lambda i: (0, 0)),
        ],
        out_specs=pl.BlockSpec((blk, o), lambda i: (i, 0)),
        out_shape=jax.ShapeDtypeStruct((b, o), jnp.float32),
    )(win, W1, b1.reshape(1, h), W2)


def kernel(x, table, W1, b1, W2):
    batch, win = x.shape
    v, d = table.shape
    xi = x.astype(jnp.int32).reshape(_NW, _NCHUNKS, _CHUNK)
    hb = _VB.bit_length() - 2
    p3d = (((xi >> (hb + 1)) << (hb + 1))
           + ((xi & ((1 << hb) - 1)) << 1)
           + ((xi >> hb) & 1))
    grid = (v + _VB - 1) // _VB
    table2 = _tc_pack(table.T, grid * (_VB // 2))
    table64 = table2.reshape(-1, _PITCH)
    packed = _sc_gather_extract(table64, p3d)
    win_emb = packed.reshape(batch, win * d)
    return _mlp(win_emb, W1, b1, W2)
